# baseline (speedup 1.0000x reference)
import numpy as np
import jax
import jax.numpy as jnp
from jax.experimental import pallas as pl
from jax.experimental.pallas import tpu as pltpu

FEAT_STRIDE = 16
NEG_OV = 0.3
POS_OV = 0.7
FG_FRAC = 0.5
RPN_BATCHSIZE = 256
LANES = 128


def _np_whctrs(a):
    w = a[2] - a[0] + 1.0
    h = a[3] - a[1] + 1.0
    return w, h, a[0] + 0.5 * (w - 1), a[1] + 0.5 * (h - 1)


def _np_mkanchors(ws, hs, xc, yc):
    ws = np.asarray(ws).reshape(-1, 1)
    hs = np.asarray(hs).reshape(-1, 1)
    return np.hstack((xc - 0.5 * (ws - 1), yc - 0.5 * (hs - 1),
                      xc + 0.5 * (ws - 1), yc + 0.5 * (hs - 1)))


def _np_base_anchors(base_size=16, ratios=(0.5, 1.0, 2.0), scales=(8.0, 16.0, 32.0)):
    ratios = np.array(ratios)
    scales = np.array(scales)
    base = np.array([1.0, 1.0, base_size, base_size]) - 1
    w, h, xc, yc = _np_whctrs(base)
    size = w * h
    ws = np.round(np.sqrt(size / ratios))
    hs = np.round(ws * ratios)
    ra = _np_mkanchors(ws, hs, xc, yc)
    out = []
    for i in range(ra.shape[0]):
        w, h, xc, yc = _np_whctrs(ra[i, :])
        out.append(_np_mkanchors(w * scales, h * scales, xc, yc))
    return np.vstack(out).astype(np.float32)


def _np_all_anchors(H, W):
    base = _np_base_anchors()
    A = base.shape[0]
    sx = np.arange(W, dtype=np.float32) * FEAT_STRIDE
    sy = np.arange(H, dtype=np.float32) * FEAT_STRIDE
    sxx, syy = np.meshgrid(sx, sy)
    shifts = np.stack([sxx.ravel(), syy.ravel(), sxx.ravel(), syy.ravel()], axis=1)
    return (base[None, :, :] + shifts[:, None, :]).reshape(-1, 4).astype(np.float32)


def _atl_kernel(G, ROWS, N):
    NUM_FG = int(FG_FRAC * RPN_BATCHSIZE)

    def body(gt_ref, imwh_ref, ax1_ref, ay1_ref, ax2_ref, ay2_ref, aarea_ref,
             valid_ref, tlane_ref, trow_ref,
             lab_ref, tx_ref, ty_ref, tw_ref, th_ref, inw_ref, outw_ref,
             mov_s, bad_s, keep_s, amax_s):
        ax1 = ax1_ref[...]
        ay1 = ay1_ref[...]
        ax2 = ax2_ref[...]
        ay2 = ay2_ref[...]
        a_area = aarea_ref[...]
        valid = valid_ref[...]

        im_h = imwh_ref[0, 0]
        im_w = imwh_ref[0, 1]
        inside = (ax1 >= 0.0) & (ay1 >= 0.0) & (ax2 < im_w) & (ay2 < im_h)

        neg_inf = jnp.float32(-3.0e38)
        mov_s[...] = jnp.full((ROWS, LANES), neg_inf, jnp.float32)
        bad_s[...] = jnp.zeros((ROWS, LANES), jnp.float32)
        keep_s[...] = jnp.zeros((ROWS, LANES), jnp.float32)
        amax_s[...] = jnp.zeros((ROWS, LANES), jnp.float32)

        def scan_inact(g, carry):
            g0f, anyf = carry
            gx1 = gt_ref[0, g, 0]
            gy1 = gt_ref[0, g, 1]
            gx2 = gt_ref[0, g, 2]
            gy2 = gt_ref[0, g, 3]
            cls = gt_ref[0, g, 4]
            nz = jnp.logical_not((gx1 == 0.0) & (gy1 == 0.0)
                                 & (gx2 == 0.0) & (gy2 == 0.0))
            inact = (cls == 2.0) | (cls == 3.0) | jnp.logical_not(nz)
            g0f = jnp.where(inact & (anyf == 0.0), g.astype(jnp.float32), g0f)
            anyf = jnp.where(inact, 1.0, anyf)
            return g0f, anyf

        g0f, anyf = jax.lax.fori_loop(0, G, scan_inact,
                                      (jnp.float32(0.0), jnp.float32(0.0)))

        def one_gt(g, carry):
            gx1 = gt_ref[0, g, 0]
            gy1 = gt_ref[0, g, 1]
            gx2 = gt_ref[0, g, 2]
            gy2 = gt_ref[0, g, 3]
            cls = gt_ref[0, g, 4]
            nz = jnp.logical_not((gx1 == 0.0) & (gy1 == 0.0)
                                 & (gx2 == 0.0) & (gy2 == 0.0))
            hard_ign = (cls == 3.0) | (cls == 2.0)
            pedv = jnp.logical_not(hard_ign) & nz

            @pl.when(pedv)
            def _():
                iw = jnp.maximum(
                    jnp.minimum(ax2, gx2) - jnp.maximum(ax1, gx1) + 1.0, 0.0)
                ih = jnp.maximum(
                    jnp.minimum(ay2, gy2) - jnp.maximum(ay1, gy1) + 1.0, 0.0)
                inter = iw * ih
                g_area = (gx2 - gx1 + 1.0) * (gy2 - gy1 + 1.0)
                iou = inter / (a_area + g_area - inter)
                ov = jnp.where(inside, iou, -1.0)
                gm = jnp.max(ov)
                gm = jnp.where(gm == 0.0, jnp.float32(1e-5), gm)
                keep_s[...] = jnp.where(ov == gm, 1.0, keep_s[...])
                max_ov = mov_s[...]
                amax_s[...] = jnp.where(ov > max_ov, g.astype(jnp.float32),
                                        amax_s[...])
                mov_s[...] = jnp.maximum(max_ov, ov)

            @pl.when(hard_ign & nz)
            def _():
                iw = jnp.maximum(
                    jnp.minimum(ax2, gx2) - jnp.maximum(ax1, gx1) + 1.0, 0.0)
                ih = jnp.maximum(
                    jnp.minimum(ay2, gy2) - jnp.maximum(ay1, gy1) + 1.0, 0.0)
                bad_s[...] = bad_s[...] + iw * ih

            return carry

        def pass1(i, carry):
            carry = one_gt(2 * i, carry)
            return one_gt(2 * i + 1, carry)

        jax.lax.fori_loop(0, G // 2, pass1, 0)
        if G % 2:
            one_gt(jnp.int32(G - 1), 0)

        pmax = mov_s[...]
        badv = bad_s[...]
        keep = keep_s[...]
        pamax = amax_s[...]

        base = jnp.where(inside, 0.0, -1.0)
        gmb = jnp.max(base)
        gmb = jnp.where(gmb == 0.0, jnp.float32(1e-5), gmb)
        anyb = anyf > 0.0
        take_inact = anyb & ((base > pmax) | ((base == pmax) & (g0f < pamax)))
        amax = jnp.where(take_inact, g0f, pamax)
        max_ov = jnp.where(anyb, jnp.maximum(pmax, base), pmax)
        keep = jnp.where(anyb & (base == gmb), 1.0, keep)

        init2 = (
            jnp.zeros((ROWS, LANES), jnp.float32),
            jnp.zeros((ROWS, LANES), jnp.float32),
            jnp.zeros((ROWS, LANES), jnp.float32),
            jnp.zeros((ROWS, LANES), jnp.float32),
        )

        def sel_gt(g, carry):
            sx1, sy1, sx2, sy2 = carry
            cls = gt_ref[0, g, 4]
            pedf = ((cls != 2.0) & (cls != 3.0)).astype(jnp.float32)
            sel = amax == g.astype(jnp.float32)
            sx1 = jnp.where(sel, gt_ref[0, g, 0] * pedf, sx1)
            sy1 = jnp.where(sel, gt_ref[0, g, 1] * pedf, sy1)
            sx2 = jnp.where(sel, gt_ref[0, g, 2] * pedf, sx2)
            sy2 = jnp.where(sel, gt_ref[0, g, 3] * pedf, sy2)
            return sx1, sy1, sx2, sy2

        def pass2(i, carry):
            carry = sel_gt(2 * i, carry)
            return sel_gt(2 * i + 1, carry)

        carry2 = jax.lax.fori_loop(0, G // 2, pass2, init2)
        if G % 2:
            carry2 = sel_gt(jnp.int32(G - 1), carry2)
        sx1, sy1, sx2, sy2 = carry2

        is_fg = (keep > 0.0) | (max_ov >= POS_OV)
        is_bg_pre = (max_ov < NEG_OV) & jnp.logical_not(is_fg)
        bad = badv > 0.0

        tlane = tlane_ref[...]
        trow = trow_ref[...]

        def prefix_rank(flags_f):
            within = jnp.dot(flags_f, tlane, preferred_element_type=jnp.float32)
            rowpref = jnp.dot(trow, flags_f, preferred_element_type=jnp.float32)
            return within + jnp.sum(rowpref, axis=1, keepdims=True)

        fg_f = jnp.where(is_fg & (valid > 0.0), 1.0, 0.0)
        fg_rank = prefix_rank(fg_f)
        total_fg = jnp.sum(fg_f)

        bg_count = is_bg_pre & jnp.logical_not(bad) & (valid > 0.0)
        bg_f = jnp.where(bg_count, 1.0, 0.0)
        bg_rank = prefix_rank(bg_f)
        num_bg = jnp.float32(RPN_BATCHSIZE) - jnp.minimum(total_fg, jnp.float32(NUM_FG))

        labels = jnp.full((ROWS, LANES), -1.0, jnp.float32)
        labels = jnp.where(bg_count & (bg_rank <= num_bg), 0.0, labels)
        labels = jnp.where(is_fg & (fg_rank <= jnp.float32(NUM_FG)), 1.0, labels)
        labels = jnp.where(inside, labels, -1.0)
        lab_ref[0] = labels

        inside_f = jnp.where(inside, 1.0, 0.0)
        ew = ax2 - ax1 + 1.0
        eh = ay2 - ay1 + 1.0
        ecx = ax1 + 0.5 * ew
        ecy = ay1 + 0.5 * eh
        gw = sx2 - sx1 + 1.0
        gh = sy2 - sy1 + 1.0
        gcx = sx1 + 0.5 * gw
        gcy = sy1 + 0.5 * gh
        tx_ref[0] = (gcx - ecx) / ew * inside_f
        ty_ref[0] = (gcy - ecy) / eh * inside_f
        tw_ref[0] = jnp.log(gw / ew) * inside_f
        th_ref[0] = jnp.log(gh / eh) * inside_f

        pos = labels == 1.0
        inw_ref[0] = jnp.where(pos, 1.0, 0.0)
        nex = jnp.sum(jnp.where((labels >= 0.0) & (valid > 0.0), 1.0, 0.0))
        pw = 1.0 / jnp.maximum(nex, 1.0)
        outw_ref[0] = jnp.where(labels >= 0.0, pw, 0.0)

    return body


def kernel(rpn_cls_score, gt_boxes, im_info, num_boxes):
    B = num_boxes.shape[0]
    H, W = rpn_cls_score.shape[2], rpn_cls_score.shape[3]
    G = gt_boxes.shape[1]
    anchors = _np_all_anchors(H, W)
    N = anchors.shape[0]
    ROWS = (N + LANES - 1) // LANES
    if ROWS % 8:
        ROWS += 8 - ROWS % 8
    NP = ROWS * LANES
    pad = NP - N
    anchors = np.concatenate([anchors, np.tile(anchors[:1], (pad, 1))], axis=0)

    ax1 = jnp.asarray(anchors[:, 0].reshape(ROWS, LANES))
    ay1 = jnp.asarray(anchors[:, 1].reshape(ROWS, LANES))
    ax2 = jnp.asarray(anchors[:, 2].reshape(ROWS, LANES))
    ay2 = jnp.asarray(anchors[:, 3].reshape(ROWS, LANES))
    a_area = jnp.asarray(
        ((anchors[:, 2] - anchors[:, 0] + 1.0)
         * (anchors[:, 3] - anchors[:, 1] + 1.0)).reshape(ROWS, LANES))
    validf = np.zeros((NP,), np.float32)
    validf[:N] = 1.0
    valid = jnp.asarray(validf.reshape(ROWS, LANES))

    tlane = jnp.asarray(np.triu(np.ones((LANES, LANES), np.float32)))
    trow = jnp.asarray(np.tril(np.ones((ROWS, ROWS), np.float32), k=-1))

    imwh = im_info[0:1, 0:2]

    grid = (B,)
    big = pl.BlockSpec((ROWS, LANES), lambda b: (0, 0))
    outspec = pl.BlockSpec((1, ROWS, LANES), lambda b: (b, 0, 0))
    outshape = jax.ShapeDtypeStruct((B, ROWS, LANES), jnp.float32)

    outs = pl.pallas_call(
        _atl_kernel(G, ROWS, N),
        grid=grid,
        in_specs=[
            pl.BlockSpec((1, G, 5), lambda b: (b, 0, 0), memory_space=pltpu.SMEM),
            pl.BlockSpec((1, 2), lambda b: (0, 0), memory_space=pltpu.SMEM),
            big, big, big, big, big, big,
            pl.BlockSpec((LANES, LANES), lambda b: (0, 0)),
            pl.BlockSpec((ROWS, ROWS), lambda b: (0, 0)),
        ],
        out_specs=[outspec] * 7,
        out_shape=[outshape] * 7,
        scratch_shapes=[pltpu.VMEM((ROWS, LANES), jnp.float32)] * 4,
        compiler_params=pltpu.CompilerParams(
            dimension_semantics=("parallel",),
        ),
    )(gt_boxes, imwh, ax1, ay1, ax2, ay2, a_area, valid, tlane, trow)

    lab, tx, ty, tw, th, inw, outw = [o.reshape(B, NP)[:, :N] for o in outs]
    labels = lab
    bbox_targets = jnp.stack([tx, ty, tw, th], axis=-1)
    ones4 = jnp.ones((1, 1, 4), jnp.float32)
    bbox_inside_w = inw[:, :, None] * ones4
    bbox_outside_w = outw[:, :, None] * ones4
    return labels, bbox_targets, bbox_inside_w, bbox_outside_w


# speedup vs baseline: 1.1491x; 1.1491x over previous
import numpy as np
import jax
import jax.numpy as jnp
from jax.experimental import pallas as pl
from jax.experimental.pallas import tpu as pltpu

FEAT_STRIDE = 16
NEG_OV = 0.3
POS_OV = 0.7
FG_FRAC = 0.5
RPN_BATCHSIZE = 256
LANES = 128


def _np_whctrs(a):
    w = a[2] - a[0] + 1.0
    h = a[3] - a[1] + 1.0
    return w, h, a[0] + 0.5 * (w - 1), a[1] + 0.5 * (h - 1)


def _np_mkanchors(ws, hs, xc, yc):
    ws = np.asarray(ws).reshape(-1, 1)
    hs = np.asarray(hs).reshape(-1, 1)
    return np.hstack((xc - 0.5 * (ws - 1), yc - 0.5 * (hs - 1),
                      xc + 0.5 * (ws - 1), yc + 0.5 * (hs - 1)))


def _np_base_anchors(base_size=16, ratios=(0.5, 1.0, 2.0), scales=(8.0, 16.0, 32.0)):
    ratios = np.array(ratios)
    scales = np.array(scales)
    base = np.array([1.0, 1.0, base_size, base_size]) - 1
    w, h, xc, yc = _np_whctrs(base)
    size = w * h
    ws = np.round(np.sqrt(size / ratios))
    hs = np.round(ws * ratios)
    ra = _np_mkanchors(ws, hs, xc, yc)
    out = []
    for i in range(ra.shape[0]):
        w, h, xc, yc = _np_whctrs(ra[i, :])
        out.append(_np_mkanchors(w * scales, h * scales, xc, yc))
    return np.vstack(out).astype(np.float32)


def _np_all_anchors(H, W):
    base = _np_base_anchors()
    A = base.shape[0]
    sx = np.arange(W, dtype=np.float32) * FEAT_STRIDE
    sy = np.arange(H, dtype=np.float32) * FEAT_STRIDE
    sxx, syy = np.meshgrid(sx, sy)
    shifts = np.stack([sxx.ravel(), syy.ravel(), sxx.ravel(), syy.ravel()], axis=1)
    return (base[None, :, :] + shifts[:, None, :]).reshape(-1, 4).astype(np.float32)


def _atl_kernel(G, ROWS, N):
    NUM_FG = int(FG_FRAC * RPN_BATCHSIZE)

    def body(gt_ref, imwh_ref, ax1_ref, ay1_ref, ax2_ref, ay2_ref, aarea_ref,
             valid_ref, tlane_ref, trow_ref,
             lab_ref, tx_ref, ty_ref, tw_ref, th_ref, inw_ref, outw_ref,
             mov_s, bad_s, keep_s, amax_s, sx1_s, sy1_s, sx2_s, sy2_s,
             g0any_s):
        ax1 = ax1_ref[...]
        ay1 = ay1_ref[...]
        ax2 = ax2_ref[...]
        ay2 = ay2_ref[...]
        a_area = aarea_ref[...]
        valid = valid_ref[...]

        im_h = imwh_ref[0, 0]
        im_w = imwh_ref[0, 1]
        inside = (ax1 >= 0.0) & (ay1 >= 0.0) & (ax2 < im_w) & (ay2 < im_h)

        neg_inf = jnp.float32(-3.0e38)
        mov_s[...] = jnp.full((ROWS, LANES), neg_inf, jnp.float32)
        bad_s[...] = jnp.zeros((ROWS, LANES), jnp.float32)
        keep_s[...] = jnp.zeros((ROWS, LANES), jnp.float32)
        amax_s[...] = jnp.zeros((ROWS, LANES), jnp.float32)

        sx1_s[...] = jnp.zeros((ROWS, LANES), jnp.float32)
        sy1_s[...] = jnp.zeros((ROWS, LANES), jnp.float32)
        sx2_s[...] = jnp.zeros((ROWS, LANES), jnp.float32)
        sy2_s[...] = jnp.zeros((ROWS, LANES), jnp.float32)
        g0any_s[0] = jnp.float32(0.0)
        g0any_s[1] = jnp.float32(0.0)

        def one_gt(g, carry):
            gx1 = gt_ref[0, g, 0]
            gy1 = gt_ref[0, g, 1]
            gx2 = gt_ref[0, g, 2]
            gy2 = gt_ref[0, g, 3]
            cls = gt_ref[0, g, 4]
            nz = jnp.logical_not((gx1 == 0.0) & (gy1 == 0.0)
                                 & (gx2 == 0.0) & (gy2 == 0.0))
            hard_ign = (cls == 3.0) | (cls == 2.0)
            pedv = jnp.logical_not(hard_ign) & nz

            inact = jnp.logical_not(pedv)
            anyv = g0any_s[1]
            g0any_s[0] = jnp.where(inact & (anyv == 0.0),
                                   g.astype(jnp.float32), g0any_s[0])
            g0any_s[1] = jnp.where(inact, 1.0, anyv)

            @pl.when(pedv)
            def _():
                iw = jnp.maximum(
                    jnp.minimum(ax2, gx2) - jnp.maximum(ax1, gx1) + 1.0, 0.0)
                ih = jnp.maximum(
                    jnp.minimum(ay2, gy2) - jnp.maximum(ay1, gy1) + 1.0, 0.0)
                inter = iw * ih
                g_area = (gx2 - gx1 + 1.0) * (gy2 - gy1 + 1.0)
                iou = inter / (a_area + g_area - inter)
                ov = jnp.where(inside, iou, -1.0)
                gm = jnp.max(ov)
                gm = jnp.where(gm == 0.0, jnp.float32(1e-5), gm)
                keep_s[...] = jnp.where(ov == gm, 1.0, keep_s[...])
                max_ov = mov_s[...]
                amax_s[...] = jnp.where(ov > max_ov, g.astype(jnp.float32),
                                        amax_s[...])
                mov_s[...] = jnp.maximum(max_ov, ov)

            @pl.when(hard_ign & nz)
            def _():
                iw = jnp.maximum(
                    jnp.minimum(ax2, gx2) - jnp.maximum(ax1, gx1) + 1.0, 0.0)
                ih = jnp.maximum(
                    jnp.minimum(ay2, gy2) - jnp.maximum(ay1, gy1) + 1.0, 0.0)
                bad_s[...] = bad_s[...] + iw * ih

            return carry

        def pass1(i, carry):
            carry = one_gt(2 * i, carry)
            return one_gt(2 * i + 1, carry)

        jax.lax.fori_loop(0, G // 2, pass1, 0)
        if G % 2:
            one_gt(jnp.int32(G - 1), 0)

        g0f = g0any_s[0]
        anyf = g0any_s[1]
        pmax = mov_s[...]
        badv = bad_s[...]
        keep = keep_s[...]
        pamax = amax_s[...]

        base = jnp.where(inside, 0.0, -1.0)
        gmb = jnp.max(base)
        gmb = jnp.where(gmb == 0.0, jnp.float32(1e-5), gmb)
        anyb = anyf > 0.0
        take_inact = anyb & ((base > pmax) | ((base == pmax) & (g0f < pamax)))
        amax = jnp.where(take_inact, g0f, pamax)
        max_ov = jnp.where(anyb, jnp.maximum(pmax, base), pmax)
        keep = jnp.where(anyb & (base == gmb), 1.0, keep)

        def sel_gt(g, carry):
            gx1 = gt_ref[0, g, 0]
            gy1 = gt_ref[0, g, 1]
            gx2 = gt_ref[0, g, 2]
            gy2 = gt_ref[0, g, 3]
            cls = gt_ref[0, g, 4]
            nz = jnp.logical_not((gx1 == 0.0) & (gy1 == 0.0)
                                 & (gx2 == 0.0) & (gy2 == 0.0))
            pedv = (cls != 2.0) & (cls != 3.0) & nz

            @pl.when(pedv)
            def _():
                sel = amax == g.astype(jnp.float32)
                sx1_s[...] = jnp.where(sel, gx1, sx1_s[...])
                sy1_s[...] = jnp.where(sel, gy1, sy1_s[...])
                sx2_s[...] = jnp.where(sel, gx2, sx2_s[...])
                sy2_s[...] = jnp.where(sel, gy2, sy2_s[...])

            return carry

        def pass2(i, carry):
            carry = sel_gt(2 * i, carry)
            return sel_gt(2 * i + 1, carry)

        jax.lax.fori_loop(0, G // 2, pass2, 0)
        if G % 2:
            sel_gt(jnp.int32(G - 1), 0)
        sx1 = sx1_s[...]
        sy1 = sy1_s[...]
        sx2 = sx2_s[...]
        sy2 = sy2_s[...]

        is_fg = (keep > 0.0) | (max_ov >= POS_OV)
        is_bg_pre = (max_ov < NEG_OV) & jnp.logical_not(is_fg)
        bad = badv > 0.0

        tlane = tlane_ref[...]
        trow = trow_ref[...]

        def prefix_rank(flags_f):
            within = jnp.dot(flags_f, tlane, preferred_element_type=jnp.float32)
            rowpref = jnp.dot(trow, flags_f, preferred_element_type=jnp.float32)
            return within + jnp.sum(rowpref, axis=1, keepdims=True)

        fg_f = jnp.where(is_fg & (valid > 0.0), 1.0, 0.0)
        fg_rank = prefix_rank(fg_f)
        total_fg = jnp.sum(fg_f)

        bg_count = is_bg_pre & jnp.logical_not(bad) & (valid > 0.0)
        bg_f = jnp.where(bg_count, 1.0, 0.0)
        bg_rank = prefix_rank(bg_f)
        num_bg = jnp.float32(RPN_BATCHSIZE) - jnp.minimum(total_fg, jnp.float32(NUM_FG))

        labels = jnp.full((ROWS, LANES), -1.0, jnp.float32)
        labels = jnp.where(bg_count & (bg_rank <= num_bg), 0.0, labels)
        labels = jnp.where(is_fg & (fg_rank <= jnp.float32(NUM_FG)), 1.0, labels)
        labels = jnp.where(inside, labels, -1.0)
        lab_ref[0] = labels

        inside_f = jnp.where(inside, 1.0, 0.0)
        ew = ax2 - ax1 + 1.0
        eh = ay2 - ay1 + 1.0
        ecx = ax1 + 0.5 * ew
        ecy = ay1 + 0.5 * eh
        gw = sx2 - sx1 + 1.0
        gh = sy2 - sy1 + 1.0
        gcx = sx1 + 0.5 * gw
        gcy = sy1 + 0.5 * gh
        tx_ref[0] = (gcx - ecx) / ew * inside_f
        ty_ref[0] = (gcy - ecy) / eh * inside_f
        tw_ref[0] = jnp.log(gw / ew) * inside_f
        th_ref[0] = jnp.log(gh / eh) * inside_f

        pos = labels == 1.0
        inw_ref[0] = jnp.where(pos, 1.0, 0.0)
        nex = jnp.sum(jnp.where((labels >= 0.0) & (valid > 0.0), 1.0, 0.0))
        pw = 1.0 / jnp.maximum(nex, 1.0)
        outw_ref[0] = jnp.where(labels >= 0.0, pw, 0.0)

    return body


def kernel(rpn_cls_score, gt_boxes, im_info, num_boxes):
    B = num_boxes.shape[0]
    H, W = rpn_cls_score.shape[2], rpn_cls_score.shape[3]
    G = gt_boxes.shape[1]
    anchors = _np_all_anchors(H, W)
    N = anchors.shape[0]
    ROWS = (N + LANES - 1) // LANES
    if ROWS % 8:
        ROWS += 8 - ROWS % 8
    NP = ROWS * LANES
    pad = NP - N
    anchors = np.concatenate([anchors, np.tile(anchors[:1], (pad, 1))], axis=0)

    ax1 = jnp.asarray(anchors[:, 0].reshape(ROWS, LANES))
    ay1 = jnp.asarray(anchors[:, 1].reshape(ROWS, LANES))
    ax2 = jnp.asarray(anchors[:, 2].reshape(ROWS, LANES))
    ay2 = jnp.asarray(anchors[:, 3].reshape(ROWS, LANES))
    a_area = jnp.asarray(
        ((anchors[:, 2] - anchors[:, 0] + 1.0)
         * (anchors[:, 3] - anchors[:, 1] + 1.0)).reshape(ROWS, LANES))
    validf = np.zeros((NP,), np.float32)
    validf[:N] = 1.0
    valid = jnp.asarray(validf.reshape(ROWS, LANES))

    tlane = jnp.asarray(np.triu(np.ones((LANES, LANES), np.float32)))
    trow = jnp.asarray(np.tril(np.ones((ROWS, ROWS), np.float32), k=-1))

    imwh = im_info[0:1, 0:2]

    grid = (B,)
    big = pl.BlockSpec((ROWS, LANES), lambda b: (0, 0))
    outspec = pl.BlockSpec((1, ROWS, LANES), lambda b: (b, 0, 0))
    outshape = jax.ShapeDtypeStruct((B, ROWS, LANES), jnp.float32)

    outs = pl.pallas_call(
        _atl_kernel(G, ROWS, N),
        grid=grid,
        in_specs=[
            pl.BlockSpec((1, G, 5), lambda b: (b, 0, 0), memory_space=pltpu.SMEM),
            pl.BlockSpec((1, 2), lambda b: (0, 0), memory_space=pltpu.SMEM),
            big, big, big, big, big, big,
            pl.BlockSpec((LANES, LANES), lambda b: (0, 0)),
            pl.BlockSpec((ROWS, ROWS), lambda b: (0, 0)),
        ],
        out_specs=[outspec] * 7,
        out_shape=[outshape] * 7,
        scratch_shapes=[pltpu.VMEM((ROWS, LANES), jnp.float32)] * 8
        + [pltpu.SMEM((2,), jnp.float32)],
        compiler_params=pltpu.CompilerParams(
            dimension_semantics=("parallel",),
        ),
    )(gt_boxes, imwh, ax1, ay1, ax2, ay2, a_area, valid, tlane, trow)

    lab, tx, ty, tw, th, inw, outw = [o.reshape(B, NP)[:, :N] for o in outs]
    labels = lab
    bbox_targets = jnp.stack([tx, ty, tw, th], axis=-1)
    ones4 = jnp.ones((1, 1, 4), jnp.float32)
    bbox_inside_w = inw[:, :, None] * ones4
    bbox_outside_w = outw[:, :, None] * ones4
    return labels, bbox_targets, bbox_inside_w, bbox_outside_w


# speedup vs baseline: 1.1659x; 1.0147x over previous
import numpy as np
import jax
import jax.numpy as jnp
from jax.experimental import pallas as pl
from jax.experimental.pallas import tpu as pltpu

FEAT_STRIDE = 16
NEG_OV = 0.3
POS_OV = 0.7
FG_FRAC = 0.5
RPN_BATCHSIZE = 256
LANES = 128


def _np_whctrs(a):
    w = a[2] - a[0] + 1.0
    h = a[3] - a[1] + 1.0
    return w, h, a[0] + 0.5 * (w - 1), a[1] + 0.5 * (h - 1)


def _np_mkanchors(ws, hs, xc, yc):
    ws = np.asarray(ws).reshape(-1, 1)
    hs = np.asarray(hs).reshape(-1, 1)
    return np.hstack((xc - 0.5 * (ws - 1), yc - 0.5 * (hs - 1),
                      xc + 0.5 * (ws - 1), yc + 0.5 * (hs - 1)))


def _np_base_anchors(base_size=16, ratios=(0.5, 1.0, 2.0), scales=(8.0, 16.0, 32.0)):
    ratios = np.array(ratios)
    scales = np.array(scales)
    base = np.array([1.0, 1.0, base_size, base_size]) - 1
    w, h, xc, yc = _np_whctrs(base)
    size = w * h
    ws = np.round(np.sqrt(size / ratios))
    hs = np.round(ws * ratios)
    ra = _np_mkanchors(ws, hs, xc, yc)
    out = []
    for i in range(ra.shape[0]):
        w, h, xc, yc = _np_whctrs(ra[i, :])
        out.append(_np_mkanchors(w * scales, h * scales, xc, yc))
    return np.vstack(out).astype(np.float32)


def _np_all_anchors(H, W):
    base = _np_base_anchors()
    A = base.shape[0]
    sx = np.arange(W, dtype=np.float32) * FEAT_STRIDE
    sy = np.arange(H, dtype=np.float32) * FEAT_STRIDE
    sxx, syy = np.meshgrid(sx, sy)
    shifts = np.stack([sxx.ravel(), syy.ravel(), sxx.ravel(), syy.ravel()], axis=1)
    return (base[None, :, :] + shifts[:, None, :]).reshape(-1, 4).astype(np.float32)


def _atl_kernel(G, ROWS, N):
    NUM_FG = int(FG_FRAC * RPN_BATCHSIZE)

    def body(gt_ref, imwh_ref, ax1_ref, ay1_ref, ax2_ref, ay2_ref, aarea_ref,
             valid_ref, tlane_ref, trow_ref,
             lab_ref, tx_ref, ty_ref, tw_ref, th_ref, inw_ref, outw_ref,
             mov_s, bad_s, keep_s, amax_s, sx1_s, sy1_s, sx2_s, sy2_s,
             g0any_s):
        ax1 = ax1_ref[...]
        ay1 = ay1_ref[...]
        ax2 = ax2_ref[...]
        ay2 = ay2_ref[...]
        a_area = aarea_ref[...]
        valid = valid_ref[...]

        im_h = imwh_ref[0, 0]
        im_w = imwh_ref[0, 1]
        inside = (ax1 >= 0.0) & (ay1 >= 0.0) & (ax2 < im_w) & (ay2 < im_h)

        neg_inf = jnp.float32(-3.0e38)
        mov_s[...] = jnp.full((ROWS, LANES), neg_inf, jnp.float32)
        bad_s[...] = jnp.zeros((ROWS, LANES), jnp.float32)
        keep_s[...] = jnp.zeros((ROWS, LANES), jnp.float32)
        amax_s[...] = jnp.zeros((ROWS, LANES), jnp.float32)

        sx1_s[...] = jnp.zeros((ROWS, LANES), jnp.float32)
        sy1_s[...] = jnp.zeros((ROWS, LANES), jnp.float32)
        sx2_s[...] = jnp.zeros((ROWS, LANES), jnp.float32)
        sy2_s[...] = jnp.zeros((ROWS, LANES), jnp.float32)
        g0any_s[0] = jnp.float32(0.0)
        g0any_s[1] = jnp.float32(0.0)

        def one_gt(g, carry):
            gx1 = gt_ref[0, g, 0]
            gy1 = gt_ref[0, g, 1]
            gx2 = gt_ref[0, g, 2]
            gy2 = gt_ref[0, g, 3]
            cls = gt_ref[0, g, 4]
            nz = jnp.logical_not((gx1 == 0.0) & (gy1 == 0.0)
                                 & (gx2 == 0.0) & (gy2 == 0.0))
            hard_ign = (cls == 3.0) | (cls == 2.0)
            pedv = jnp.logical_not(hard_ign) & nz

            inact = jnp.logical_not(pedv)
            anyv = g0any_s[1]
            g0any_s[0] = jnp.where(inact & (anyv == 0.0),
                                   g.astype(jnp.float32), g0any_s[0])
            g0any_s[1] = jnp.where(inact, 1.0, anyv)

            @pl.when(pedv)
            def _():
                iw = jnp.maximum(
                    jnp.minimum(ax2, gx2) - jnp.maximum(ax1, gx1) + 1.0, 0.0)
                ih = jnp.maximum(
                    jnp.minimum(ay2, gy2) - jnp.maximum(ay1, gy1) + 1.0, 0.0)
                inter = iw * ih
                g_area = (gx2 - gx1 + 1.0) * (gy2 - gy1 + 1.0)
                iou = inter / (a_area + g_area - inter)
                ov = jnp.where(inside, iou, -1.0)
                gm = jnp.max(ov)
                gm = jnp.where(gm == 0.0, jnp.float32(1e-5), gm)
                keep_s[...] = jnp.where(ov == gm, 1.0, keep_s[...])
                max_ov = mov_s[...]
                amax_s[...] = jnp.where(ov > max_ov, g.astype(jnp.float32),
                                        amax_s[...])
                mov_s[...] = jnp.maximum(max_ov, ov)

            @pl.when(hard_ign & nz)
            def _():
                iw = jnp.maximum(
                    jnp.minimum(ax2, gx2) - jnp.maximum(ax1, gx1) + 1.0, 0.0)
                ih = jnp.maximum(
                    jnp.minimum(ay2, gy2) - jnp.maximum(ay1, gy1) + 1.0, 0.0)
                bad_s[...] = bad_s[...] + iw * ih

            return carry

        UN = 5 if G % 5 == 0 else 2
        def pass1(i, carry):
            for u in range(UN):
                carry = one_gt(UN * i + u, carry)
            return carry

        jax.lax.fori_loop(0, G // UN, pass1, 0)
        for r in range(G - (G // UN) * UN):
            one_gt(jnp.int32((G // UN) * UN + r), 0)

        g0f = g0any_s[0]
        anyf = g0any_s[1]
        pmax = mov_s[...]
        badv = bad_s[...]
        keep = keep_s[...]
        pamax = amax_s[...]

        base = jnp.where(inside, 0.0, -1.0)
        gmb = jnp.max(base)
        gmb = jnp.where(gmb == 0.0, jnp.float32(1e-5), gmb)
        anyb = anyf > 0.0
        take_inact = anyb & ((base > pmax) | ((base == pmax) & (g0f < pamax)))
        amax = jnp.where(take_inact, g0f, pamax)
        max_ov = jnp.where(anyb, jnp.maximum(pmax, base), pmax)
        keep = jnp.where(anyb & (base == gmb), 1.0, keep)

        def sel_gt(g, carry):
            gx1 = gt_ref[0, g, 0]
            gy1 = gt_ref[0, g, 1]
            gx2 = gt_ref[0, g, 2]
            gy2 = gt_ref[0, g, 3]
            cls = gt_ref[0, g, 4]
            nz = jnp.logical_not((gx1 == 0.0) & (gy1 == 0.0)
                                 & (gx2 == 0.0) & (gy2 == 0.0))
            pedv = (cls != 2.0) & (cls != 3.0) & nz

            @pl.when(pedv)
            def _():
                sel = amax == g.astype(jnp.float32)
                sx1_s[...] = jnp.where(sel, gx1, sx1_s[...])
                sy1_s[...] = jnp.where(sel, gy1, sy1_s[...])
                sx2_s[...] = jnp.where(sel, gx2, sx2_s[...])
                sy2_s[...] = jnp.where(sel, gy2, sy2_s[...])

            return carry

        def pass2(i, carry):
            for u in range(UN):
                carry = sel_gt(UN * i + u, carry)
            return carry

        jax.lax.fori_loop(0, G // UN, pass2, 0)
        for r in range(G - (G // UN) * UN):
            sel_gt(jnp.int32((G // UN) * UN + r), 0)
        sx1 = sx1_s[...]
        sy1 = sy1_s[...]
        sx2 = sx2_s[...]
        sy2 = sy2_s[...]

        is_fg = (keep > 0.0) | (max_ov >= POS_OV)
        is_bg_pre = (max_ov < NEG_OV) & jnp.logical_not(is_fg)
        bad = badv > 0.0

        tlane = tlane_ref[...]
        trow = trow_ref[...]

        def prefix_rank(flags_f):
            within = jnp.dot(flags_f, tlane, preferred_element_type=jnp.float32)
            rowpref = jnp.dot(trow, flags_f, preferred_element_type=jnp.float32)
            return within + jnp.sum(rowpref, axis=1, keepdims=True)

        fg_f = jnp.where(is_fg & (valid > 0.0), 1.0, 0.0)
        fg_rank = prefix_rank(fg_f)
        total_fg = jnp.sum(fg_f)

        bg_count = is_bg_pre & jnp.logical_not(bad) & (valid > 0.0)
        bg_f = jnp.where(bg_count, 1.0, 0.0)
        bg_rank = prefix_rank(bg_f)
        num_bg = jnp.float32(RPN_BATCHSIZE) - jnp.minimum(total_fg, jnp.float32(NUM_FG))

        labels = jnp.full((ROWS, LANES), -1.0, jnp.float32)
        labels = jnp.where(bg_count & (bg_rank <= num_bg), 0.0, labels)
        labels = jnp.where(is_fg & (fg_rank <= jnp.float32(NUM_FG)), 1.0, labels)
        labels = jnp.where(inside, labels, -1.0)
        lab_ref[0] = labels

        inside_f = jnp.where(inside, 1.0, 0.0)
        ew = ax2 - ax1 + 1.0
        eh = ay2 - ay1 + 1.0
        ecx = ax1 + 0.5 * ew
        ecy = ay1 + 0.5 * eh
        gw = sx2 - sx1 + 1.0
        gh = sy2 - sy1 + 1.0
        gcx = sx1 + 0.5 * gw
        gcy = sy1 + 0.5 * gh
        tx_ref[0] = (gcx - ecx) / ew * inside_f
        ty_ref[0] = (gcy - ecy) / eh * inside_f
        tw_ref[0] = jnp.log(gw / ew) * inside_f
        th_ref[0] = jnp.log(gh / eh) * inside_f

        pos = labels == 1.0
        inw_ref[0] = jnp.where(pos, 1.0, 0.0)
        nex = jnp.sum(jnp.where((labels >= 0.0) & (valid > 0.0), 1.0, 0.0))
        pw = 1.0 / jnp.maximum(nex, 1.0)
        outw_ref[0] = jnp.where(labels >= 0.0, pw, 0.0)

    return body


def kernel(rpn_cls_score, gt_boxes, im_info, num_boxes):
    B = num_boxes.shape[0]
    H, W = rpn_cls_score.shape[2], rpn_cls_score.shape[3]
    G = gt_boxes.shape[1]
    anchors = _np_all_anchors(H, W)
    N = anchors.shape[0]
    ROWS = (N + LANES - 1) // LANES
    if ROWS % 8:
        ROWS += 8 - ROWS % 8
    NP = ROWS * LANES
    pad = NP - N
    anchors = np.concatenate([anchors, np.tile(anchors[:1], (pad, 1))], axis=0)

    ax1 = jnp.asarray(anchors[:, 0].reshape(ROWS, LANES))
    ay1 = jnp.asarray(anchors[:, 1].reshape(ROWS, LANES))
    ax2 = jnp.asarray(anchors[:, 2].reshape(ROWS, LANES))
    ay2 = jnp.asarray(anchors[:, 3].reshape(ROWS, LANES))
    a_area = jnp.asarray(
        ((anchors[:, 2] - anchors[:, 0] + 1.0)
         * (anchors[:, 3] - anchors[:, 1] + 1.0)).reshape(ROWS, LANES))
    validf = np.zeros((NP,), np.float32)
    validf[:N] = 1.0
    valid = jnp.asarray(validf.reshape(ROWS, LANES))

    tlane = jnp.asarray(np.triu(np.ones((LANES, LANES), np.float32)))
    trow = jnp.asarray(np.tril(np.ones((ROWS, ROWS), np.float32), k=-1))

    imwh = im_info[0:1, 0:2]

    grid = (B,)
    big = pl.BlockSpec((ROWS, LANES), lambda b: (0, 0))
    outspec = pl.BlockSpec((1, ROWS, LANES), lambda b: (b, 0, 0))
    outshape = jax.ShapeDtypeStruct((B, ROWS, LANES), jnp.float32)

    outs = pl.pallas_call(
        _atl_kernel(G, ROWS, N),
        grid=grid,
        in_specs=[
            pl.BlockSpec((1, G, 5), lambda b: (b, 0, 0), memory_space=pltpu.SMEM),
            pl.BlockSpec((1, 2), lambda b: (0, 0), memory_space=pltpu.SMEM),
            big, big, big, big, big, big,
            pl.BlockSpec((LANES, LANES), lambda b: (0, 0)),
            pl.BlockSpec((ROWS, ROWS), lambda b: (0, 0)),
        ],
        out_specs=[outspec] * 7,
        out_shape=[outshape] * 7,
        scratch_shapes=[pltpu.VMEM((ROWS, LANES), jnp.float32)] * 8
        + [pltpu.SMEM((2,), jnp.float32)],
        compiler_params=pltpu.CompilerParams(
            dimension_semantics=("parallel",),
        ),
    )(gt_boxes, imwh, ax1, ay1, ax2, ay2, a_area, valid, tlane, trow)

    lab, tx, ty, tw, th, inw, outw = [o.reshape(B, NP)[:, :N] for o in outs]
    labels = lab
    bbox_targets = jnp.stack([tx, ty, tw, th], axis=-1)
    ones4 = jnp.ones((1, 1, 4), jnp.float32)
    bbox_inside_w = inw[:, :, None] * ones4
    bbox_outside_w = outw[:, :, None] * ones4
    return labels, bbox_targets, bbox_inside_w, bbox_outside_w


# speedup vs baseline: 1.1725x; 1.0057x over previous
import numpy as np
import jax
import jax.numpy as jnp
from jax.experimental import pallas as pl
from jax.experimental.pallas import tpu as pltpu

FEAT_STRIDE = 16
NEG_OV = 0.3
POS_OV = 0.7
FG_FRAC = 0.5
RPN_BATCHSIZE = 256
LANES = 128


def _np_whctrs(a):
    w = a[2] - a[0] + 1.0
    h = a[3] - a[1] + 1.0
    return w, h, a[0] + 0.5 * (w - 1), a[1] + 0.5 * (h - 1)


def _np_mkanchors(ws, hs, xc, yc):
    ws = np.asarray(ws).reshape(-1, 1)
    hs = np.asarray(hs).reshape(-1, 1)
    return np.hstack((xc - 0.5 * (ws - 1), yc - 0.5 * (hs - 1),
                      xc + 0.5 * (ws - 1), yc + 0.5 * (hs - 1)))


def _np_base_anchors(base_size=16, ratios=(0.5, 1.0, 2.0), scales=(8.0, 16.0, 32.0)):
    ratios = np.array(ratios)
    scales = np.array(scales)
    base = np.array([1.0, 1.0, base_size, base_size]) - 1
    w, h, xc, yc = _np_whctrs(base)
    size = w * h
    ws = np.round(np.sqrt(size / ratios))
    hs = np.round(ws * ratios)
    ra = _np_mkanchors(ws, hs, xc, yc)
    out = []
    for i in range(ra.shape[0]):
        w, h, xc, yc = _np_whctrs(ra[i, :])
        out.append(_np_mkanchors(w * scales, h * scales, xc, yc))
    return np.vstack(out).astype(np.float32)


def _np_all_anchors(H, W):
    base = _np_base_anchors()
    A = base.shape[0]
    sx = np.arange(W, dtype=np.float32) * FEAT_STRIDE
    sy = np.arange(H, dtype=np.float32) * FEAT_STRIDE
    sxx, syy = np.meshgrid(sx, sy)
    shifts = np.stack([sxx.ravel(), syy.ravel(), sxx.ravel(), syy.ravel()], axis=1)
    return (base[None, :, :] + shifts[:, None, :]).reshape(-1, 4).astype(np.float32)


def _atl_kernel(G, ROWS, N):
    NUM_FG = int(FG_FRAC * RPN_BATCHSIZE)

    def body(gt_ref, imwh_ref, ax1_ref, ay1_ref, ax2_ref, ay2_ref, aarea_ref,
             valid_ref, tlane_ref, trow_ref,
             lab_ref, tx_ref, ty_ref, tw_ref, th_ref, inw_ref, outw_ref,
             mov_s, bad_s, keep_s, amax_s, sx1_s, sy1_s, sx2_s, sy2_s,
             g0any_s):
        ax1 = ax1_ref[...]
        ay1 = ay1_ref[...]
        ax2 = ax2_ref[...]
        ay2 = ay2_ref[...]
        a_area = aarea_ref[...]
        valid = valid_ref[...]

        im_h = imwh_ref[0, 0]
        im_w = imwh_ref[0, 1]
        inside = (ax1 >= 0.0) & (ay1 >= 0.0) & (ax2 < im_w) & (ay2 < im_h)

        neg_inf = jnp.float32(-3.0e38)
        mov_s[...] = jnp.full((ROWS, LANES), neg_inf, jnp.float32)
        bad_s[...] = jnp.zeros((ROWS, LANES), jnp.float32)
        keep_s[...] = jnp.zeros((ROWS, LANES), jnp.float32)
        amax_s[...] = jnp.zeros((ROWS, LANES), jnp.float32)

        sx1_s[...] = jnp.zeros((ROWS, LANES), jnp.float32)
        sy1_s[...] = jnp.zeros((ROWS, LANES), jnp.float32)
        sx2_s[...] = jnp.zeros((ROWS, LANES), jnp.float32)
        sy2_s[...] = jnp.zeros((ROWS, LANES), jnp.float32)
        g0any_s[0] = jnp.float32(0.0)
        g0any_s[1] = jnp.float32(0.0)

        def one_gt(g, carry):
            gx1 = gt_ref[0, g, 0]
            gy1 = gt_ref[0, g, 1]
            gx2 = gt_ref[0, g, 2]
            gy2 = gt_ref[0, g, 3]
            cls = gt_ref[0, g, 4]
            nz = jnp.logical_not((gx1 == 0.0) & (gy1 == 0.0)
                                 & (gx2 == 0.0) & (gy2 == 0.0))
            hard_ign = (cls == 3.0) | (cls == 2.0)
            pedv = jnp.logical_not(hard_ign) & nz

            inact = jnp.logical_not(pedv)
            anyv = g0any_s[1]
            g0any_s[0] = jnp.where(inact & (anyv == 0.0),
                                   g.astype(jnp.float32), g0any_s[0])
            g0any_s[1] = jnp.where(inact, 1.0, anyv)

            @pl.when(pedv)
            def _():
                iw = jnp.maximum(
                    jnp.minimum(ax2, gx2) - jnp.maximum(ax1, gx1) + 1.0, 0.0)
                ih = jnp.maximum(
                    jnp.minimum(ay2, gy2) - jnp.maximum(ay1, gy1) + 1.0, 0.0)
                inter = iw * ih
                g_area = (gx2 - gx1 + 1.0) * (gy2 - gy1 + 1.0)
                iou = inter / (a_area + g_area - inter)
                ov = jnp.where(inside, iou, -1.0)
                gm = jnp.max(ov)
                gm = jnp.where(gm == 0.0, jnp.float32(1e-5), gm)
                keep_s[...] = jnp.where(ov == gm, 1.0, keep_s[...])
                max_ov = mov_s[...]
                amax_s[...] = jnp.where(ov > max_ov, g.astype(jnp.float32),
                                        amax_s[...])
                mov_s[...] = jnp.maximum(max_ov, ov)

            @pl.when(hard_ign & nz)
            def _():
                iw = jnp.maximum(
                    jnp.minimum(ax2, gx2) - jnp.maximum(ax1, gx1) + 1.0, 0.0)
                ih = jnp.maximum(
                    jnp.minimum(ay2, gy2) - jnp.maximum(ay1, gy1) + 1.0, 0.0)
                bad_s[...] = bad_s[...] + iw * ih

            return carry

        UN = 10 if G % 10 == 0 else (5 if G % 5 == 0 else 2)
        def pass1(i, carry):
            for u in range(UN):
                carry = one_gt(UN * i + u, carry)
            return carry

        jax.lax.fori_loop(0, G // UN, pass1, 0)
        for r in range(G - (G // UN) * UN):
            one_gt(jnp.int32((G // UN) * UN + r), 0)

        g0f = g0any_s[0]
        anyf = g0any_s[1]
        pmax = mov_s[...]
        badv = bad_s[...]
        keep = keep_s[...]
        pamax = amax_s[...]

        base = jnp.where(inside, 0.0, -1.0)
        gmb = jnp.max(base)
        gmb = jnp.where(gmb == 0.0, jnp.float32(1e-5), gmb)
        anyb = anyf > 0.0
        take_inact = anyb & ((base > pmax) | ((base == pmax) & (g0f < pamax)))
        amax = jnp.where(take_inact, g0f, pamax)
        max_ov = jnp.where(anyb, jnp.maximum(pmax, base), pmax)
        keep = jnp.where(anyb & (base == gmb), 1.0, keep)

        def sel_gt(g, carry):
            gx1 = gt_ref[0, g, 0]
            gy1 = gt_ref[0, g, 1]
            gx2 = gt_ref[0, g, 2]
            gy2 = gt_ref[0, g, 3]
            cls = gt_ref[0, g, 4]
            nz = jnp.logical_not((gx1 == 0.0) & (gy1 == 0.0)
                                 & (gx2 == 0.0) & (gy2 == 0.0))
            pedv = (cls != 2.0) & (cls != 3.0) & nz

            @pl.when(pedv)
            def _():
                sel = amax == g.astype(jnp.float32)
                sx1_s[...] = jnp.where(sel, gx1, sx1_s[...])
                sy1_s[...] = jnp.where(sel, gy1, sy1_s[...])
                sx2_s[...] = jnp.where(sel, gx2, sx2_s[...])
                sy2_s[...] = jnp.where(sel, gy2, sy2_s[...])

            return carry

        def pass2(i, carry):
            for u in range(UN):
                carry = sel_gt(UN * i + u, carry)
            return carry

        jax.lax.fori_loop(0, G // UN, pass2, 0)
        for r in range(G - (G // UN) * UN):
            sel_gt(jnp.int32((G // UN) * UN + r), 0)
        sx1 = sx1_s[...]
        sy1 = sy1_s[...]
        sx2 = sx2_s[...]
        sy2 = sy2_s[...]

        is_fg = (keep > 0.0) | (max_ov >= POS_OV)
        is_bg_pre = (max_ov < NEG_OV) & jnp.logical_not(is_fg)
        bad = badv > 0.0

        tlane = tlane_ref[...]
        trow = trow_ref[...]

        def prefix_rank(flags_f):
            within = jnp.dot(flags_f, tlane, preferred_element_type=jnp.float32)
            rowpref = jnp.dot(trow, flags_f, preferred_element_type=jnp.float32)
            return within + jnp.sum(rowpref, axis=1, keepdims=True)

        fg_f = jnp.where(is_fg & (valid > 0.0), 1.0, 0.0)
        fg_rank = prefix_rank(fg_f)
        total_fg = jnp.sum(fg_f)

        bg_count = is_bg_pre & jnp.logical_not(bad) & (valid > 0.0)
        bg_f = jnp.where(bg_count, 1.0, 0.0)
        bg_rank = prefix_rank(bg_f)
        num_bg = jnp.float32(RPN_BATCHSIZE) - jnp.minimum(total_fg, jnp.float32(NUM_FG))

        labels = jnp.full((ROWS, LANES), -1.0, jnp.float32)
        labels = jnp.where(bg_count & (bg_rank <= num_bg), 0.0, labels)
        labels = jnp.where(is_fg & (fg_rank <= jnp.float32(NUM_FG)), 1.0, labels)
        labels = jnp.where(inside, labels, -1.0)
        lab_ref[0] = labels

        inside_f = jnp.where(inside, 1.0, 0.0)
        ew = ax2 - ax1 + 1.0
        eh = ay2 - ay1 + 1.0
        ecx = ax1 + 0.5 * ew
        ecy = ay1 + 0.5 * eh
        gw = sx2 - sx1 + 1.0
        gh = sy2 - sy1 + 1.0
        gcx = sx1 + 0.5 * gw
        gcy = sy1 + 0.5 * gh
        tx_ref[0] = (gcx - ecx) / ew * inside_f
        ty_ref[0] = (gcy - ecy) / eh * inside_f
        tw_ref[0] = jnp.log(gw / ew) * inside_f
        th_ref[0] = jnp.log(gh / eh) * inside_f

        pos = labels == 1.0
        inw_ref[0] = jnp.where(pos, 1.0, 0.0)
        nex = jnp.sum(jnp.where((labels >= 0.0) & (valid > 0.0), 1.0, 0.0))
        pw = 1.0 / jnp.maximum(nex, 1.0)
        outw_ref[0] = jnp.where(labels >= 0.0, pw, 0.0)

    return body


def kernel(rpn_cls_score, gt_boxes, im_info, num_boxes):
    B = num_boxes.shape[0]
    H, W = rpn_cls_score.shape[2], rpn_cls_score.shape[3]
    G = gt_boxes.shape[1]
    anchors = _np_all_anchors(H, W)
    N = anchors.shape[0]
    ROWS = (N + LANES - 1) // LANES
    if ROWS % 8:
        ROWS += 8 - ROWS % 8
    NP = ROWS * LANES
    pad = NP - N
    anchors = np.concatenate([anchors, np.tile(anchors[:1], (pad, 1))], axis=0)

    ax1 = jnp.asarray(anchors[:, 0].reshape(ROWS, LANES))
    ay1 = jnp.asarray(anchors[:, 1].reshape(ROWS, LANES))
    ax2 = jnp.asarray(anchors[:, 2].reshape(ROWS, LANES))
    ay2 = jnp.asarray(anchors[:, 3].reshape(ROWS, LANES))
    a_area = jnp.asarray(
        ((anchors[:, 2] - anchors[:, 0] + 1.0)
         * (anchors[:, 3] - anchors[:, 1] + 1.0)).reshape(ROWS, LANES))
    validf = np.zeros((NP,), np.float32)
    validf[:N] = 1.0
    valid = jnp.asarray(validf.reshape(ROWS, LANES))

    tlane = jnp.asarray(np.triu(np.ones((LANES, LANES), np.float32)))
    trow = jnp.asarray(np.tril(np.ones((ROWS, ROWS), np.float32), k=-1))

    imwh = im_info[0:1, 0:2]

    grid = (B,)
    big = pl.BlockSpec((ROWS, LANES), lambda b: (0, 0))
    outspec = pl.BlockSpec((1, ROWS, LANES), lambda b: (b, 0, 0))
    outshape = jax.ShapeDtypeStruct((B, ROWS, LANES), jnp.float32)

    outs = pl.pallas_call(
        _atl_kernel(G, ROWS, N),
        grid=grid,
        in_specs=[
            pl.BlockSpec((1, G, 5), lambda b: (b, 0, 0), memory_space=pltpu.SMEM),
            pl.BlockSpec((1, 2), lambda b: (0, 0), memory_space=pltpu.SMEM),
            big, big, big, big, big, big,
            pl.BlockSpec((LANES, LANES), lambda b: (0, 0)),
            pl.BlockSpec((ROWS, ROWS), lambda b: (0, 0)),
        ],
        out_specs=[outspec] * 7,
        out_shape=[outshape] * 7,
        scratch_shapes=[pltpu.VMEM((ROWS, LANES), jnp.float32)] * 8
        + [pltpu.SMEM((2,), jnp.float32)],
        compiler_params=pltpu.CompilerParams(
            dimension_semantics=("parallel",),
        ),
    )(gt_boxes, imwh, ax1, ay1, ax2, ay2, a_area, valid, tlane, trow)

    lab, tx, ty, tw, th, inw, outw = [o.reshape(B, NP)[:, :N] for o in outs]
    labels = lab
    bbox_targets = jnp.stack([tx, ty, tw, th], axis=-1)
    ones4 = jnp.ones((1, 1, 4), jnp.float32)
    bbox_inside_w = inw[:, :, None] * ones4
    bbox_outside_w = outw[:, :, None] * ones4
    return labels, bbox_targets, bbox_inside_w, bbox_outside_w
